# baseline (speedup 1.0000x reference)
import functools

import jax
import jax.numpy as jnp
from jax import lax
from jax.experimental import pallas as pl
from jax.experimental.pallas import tpu as pltpu
from jax.experimental.pallas import tpu_sc as plsc

H = 128
EPS = 1e-12

NC, NS = 2, 16
NW = NC * NS
N_TOK = 1024 * 50
N_SLICES = 1
S_TOK = N_TOK // N_SLICES
CHUNK = 80

T_BLK = 1600


def _sc_gather(idx, e0, e1, e2p):
    mesh = plsc.VectorSubcoreMesh(core_axis_name="c", subcore_axis_name="s")
    f0, f1 = e0.shape[1], e1.shape[1]
    n_tok = idx.shape[0]
    b_per_w = n_tok // NW
    n_chunks = b_per_w // CHUNK

    @functools.partial(
        pl.kernel,
        mesh=mesh,
        out_type=[
            jax.ShapeDtypeStruct((n_tok, f0), jnp.float32),
            jax.ShapeDtypeStruct((n_tok, f1), jnp.float32),
            jax.ShapeDtypeStruct((n_tok, 128), jnp.float32),
        ],
        scratch_types=[
            pltpu.VMEM((b_per_w,), jnp.int32),
            pltpu.VMEM((CHUNK, f0), jnp.float32),
            pltpu.VMEM((CHUNK, f1), jnp.float32),
            pltpu.VMEM((CHUNK, 128), jnp.float32),
            pltpu.VMEM((CHUNK, f0), jnp.float32),
            pltpu.VMEM((CHUNK, f1), jnp.float32),
            pltpu.VMEM((CHUNK, 128), jnp.float32),
            pltpu.SemaphoreType.DMA,
            pltpu.SemaphoreType.DMA,
            pltpu.SemaphoreType.DMA,
            pltpu.SemaphoreType.DMA,
        ],
    )
    def k(idx_hbm, t0, t1, t2, o0, o1, o2, idx_v, r0a, r1a, r2a, r0b, r1b,
          r2b, sga, sgb, swa, swb):
        wid = lax.axis_index("s") * NC + lax.axis_index("c")
        base0 = wid * b_per_w
        pltpu.sync_copy(idx_hbm.at[pl.ds(base0, b_per_w)], idx_v)

        def start_gather(c, r0, r1, r2, sg):
            iv = idx_v.at[pl.ds(c * CHUNK, CHUNK)]
            pltpu.async_copy(t0.at[iv], r0, sg)
            pltpu.async_copy(t1.at[iv], r1, sg)
            pltpu.async_copy(t2.at[iv], r2, sg)

        def wait_gather(r0, r1, r2, sg):
            iv = idx_v.at[pl.ds(0, CHUNK)]
            pltpu.make_async_copy(t0.at[iv], r0, sg).wait()
            pltpu.make_async_copy(t1.at[iv], r1, sg).wait()
            pltpu.make_async_copy(t2.at[iv], r2, sg).wait()

        def start_wb(c, r0, r1, r2, sw):
            base = base0 + c * CHUNK
            pltpu.async_copy(r0, o0.at[pl.ds(base, CHUNK)], sw)
            pltpu.async_copy(r1, o1.at[pl.ds(base, CHUNK)], sw)
            pltpu.async_copy(r2, o2.at[pl.ds(base, CHUNK)], sw)

        def wait_wb(r0, r1, r2, sw):
            pltpu.make_async_copy(r0, o0.at[pl.ds(0, CHUNK)], sw).wait()
            pltpu.make_async_copy(r1, o1.at[pl.ds(0, CHUNK)], sw).wait()
            pltpu.make_async_copy(r2, o2.at[pl.ds(0, CHUNK)], sw).wait()

        start_gather(0, r0a, r1a, r2a, sga)

        @pl.loop(0, n_chunks, step=2)
        def _(c):
            start_gather(c + 1, r0b, r1b, r2b, sgb)
            wait_gather(r0a, r1a, r2a, sga)
            start_wb(c, r0a, r1a, r2a, swa)
            wait_wb(r0a, r1a, r2a, swa)

            @pl.when(c + 2 < n_chunks)
            def _():
                start_gather(c + 2, r0a, r1a, r2a, sga)

            wait_gather(r0b, r1b, r2b, sgb)
            start_wb(c + 1, r0b, r1b, r2b, swb)
            wait_wb(r0b, r1b, r2b, swb)

    return k(idx, e0, e1, e2p)


N_SUB = 8


def _tc_body(*refs):
    g0, g1, g2p, w0, w1, w2, b0, b1, b2, aw, ab, pr, lng, lnb = refs[-15:-1]
    out = refs[-1]
    rows = T_BLK // N_SUB

    def chain(r0):
        sl = pl.ds(r0, rows)
        h0 = jnp.dot(g0[sl, :], w0[...],
                     preferred_element_type=jnp.float32) + b0[...]
        h1 = jnp.dot(g1[sl, :], w1[...],
                     preferred_element_type=jnp.float32) + b1[...]
        h2 = jnp.dot(g2p[sl, :64], w2[...],
                     preferred_element_type=jnp.float32) + b2[...]
        s = (
            jnp.dot(jnp.tanh(h0), aw[0:H, :],
                    preferred_element_type=jnp.float32)
            + jnp.dot(jnp.tanh(h1), aw[H:2 * H, :],
                      preferred_element_type=jnp.float32)
            + jnp.dot(jnp.tanh(h2), aw[2 * H:3 * H, :],
                      preferred_element_type=jnp.float32)
            + ab[...]
        )
        e = jnp.exp(s)
        p = e / jnp.sum(e, axis=1, keepdims=True)
        x = h0 * p[:, 0:1] + h1 * p[:, 1:2] + h2 * p[:, 2:3] \
            + pr[sl, :]
        mu = jnp.mean(x, axis=1, keepdims=True)
        xc = x - mu
        var = jnp.mean(xc * xc, axis=1, keepdims=True)
        return xc * lax.rsqrt(var + EPS) * lng[...] + lnb[...]

    ys = [chain(i * rows) for i in range(N_SUB)]
    seqs = rows // 50
    for i in range(N_SUB):
        out[i * seqs:(i + 1) * seqs] = ys[i].reshape(seqs, 50, H)


def _tc_fuse(g0, g1, g2p, W0, W1, W2, b0, b1, b2, attn_W, attn_b, posrole,
             ln_g, ln_b, out_buf=None, blk_off=0, out_rows=None):
    f0, f1, f2 = g0.shape[1], g1.shape[1], W2.shape[0]
    n_tok = g0.shape[0]
    if out_rows is None:
        out_rows = n_tok // 50
    blk = lambda i: (i, 0)
    rep = lambda i: (0, 0)
    extra_in, extra_specs, alias = (), [], {}
    if out_buf is not None:
        extra_in = (out_buf,)
        extra_specs = [pl.BlockSpec(memory_space=pl.ANY)]
        alias = {0: 0}
    return pl.pallas_call(
        _tc_body,
        grid=(n_tok // T_BLK,),
        input_output_aliases=alias,
        in_specs=extra_specs + [
            pl.BlockSpec((T_BLK, f0), blk),
            pl.BlockSpec((T_BLK, f1), blk),
            pl.BlockSpec((T_BLK, 128), blk),
            pl.BlockSpec((f0, H), rep),
            pl.BlockSpec((f1, H), rep),
            pl.BlockSpec((f2, H), rep),
            pl.BlockSpec((1, H), rep),
            pl.BlockSpec((1, H), rep),
            pl.BlockSpec((1, H), rep),
            pl.BlockSpec((3 * H, 3), rep),
            pl.BlockSpec((1, 3), rep),
            pl.BlockSpec((T_BLK, H), rep),
            pl.BlockSpec((1, H), rep),
            pl.BlockSpec((1, H), rep),
        ],
        out_specs=pl.BlockSpec((T_BLK // 50, 50, H),
                               lambda i: (i + blk_off, 0, 0)),
        out_shape=jax.ShapeDtypeStruct((out_rows, 50, H), jnp.float32),
    )(*extra_in, g0, g1, g2p, W0, W1, W2, b0, b1, b2, attn_W, attn_b,
      posrole, ln_g, ln_b)


def kernel(node_ids, emb0, emb1, emb2, W0, W1, W2, b0, b1, b2, pos_table,
           role_table, attn_W, attn_b, ln_g, ln_b):
    B, S = node_ids.shape
    idx = node_ids.reshape(-1).astype(jnp.int32)
    e2p = jnp.pad(emb2, ((0, 0), (0, 128 - emb2.shape[1])))

    role_ids = jnp.ones((S,), dtype=jnp.int32).at[0].set(0)
    posrole = pos_table[:S] + role_table[role_ids]
    posrole = jnp.tile(posrole, (T_BLK // S, 1))

    b0r, b1r, b2r = b0.reshape(1, H), b1.reshape(1, H), b2.reshape(1, H)
    abr = attn_b.reshape(1, 3)
    lngr, lnbr = ln_g.reshape(1, H), ln_b.reshape(1, H)

    out = None
    blks_per_slice = S_TOK // T_BLK
    for si in range(N_SLICES):
        isl = lax.slice(idx, (si * S_TOK,), ((si + 1) * S_TOK,))
        g0, g1, g2p = _sc_gather(isl, emb0, emb1, e2p)
        out = _tc_fuse(g0, g1, g2p, W0, W1, W2, b0r, b1r, b2r,
                       attn_W, abr, posrole, lngr, lnbr,
                       out_buf=out, blk_off=si * blks_per_slice,
                       out_rows=B)
    return out


# speedup vs baseline: 1.0688x; 1.0688x over previous
import functools

import jax
import jax.numpy as jnp
from jax import lax
from jax.experimental import pallas as pl
from jax.experimental.pallas import tpu as pltpu
from jax.experimental.pallas import tpu_sc as plsc

H = 128
EPS = 1e-12

NC, NS = 2, 16
NW = NC * NS
N_TOK = 1024 * 50
N_SLICES = 2
S_TOK = N_TOK // N_SLICES
CHUNK = 80

T_BLK = 1600


def _sc_gather(idx, e0, e1, e2p):
    mesh = plsc.VectorSubcoreMesh(core_axis_name="c", subcore_axis_name="s")
    f0, f1 = e0.shape[1], e1.shape[1]
    n_tok = idx.shape[0]
    b_per_w = n_tok // NW
    n_chunks = b_per_w // CHUNK
    assert n_chunks % 2 == 0 and n_chunks * CHUNK == b_per_w

    @functools.partial(
        pl.kernel,
        mesh=mesh,
        out_type=[
            jax.ShapeDtypeStruct((n_tok, f0), jnp.float32),
            jax.ShapeDtypeStruct((n_tok, f1), jnp.float32),
            jax.ShapeDtypeStruct((n_tok, 128), jnp.float32),
        ],
        scratch_types=[
            pltpu.VMEM((b_per_w,), jnp.int32),
            pltpu.VMEM((CHUNK, f0), jnp.float32),
            pltpu.VMEM((CHUNK, f1), jnp.float32),
            pltpu.VMEM((CHUNK, 128), jnp.float32),
            pltpu.VMEM((CHUNK, f0), jnp.float32),
            pltpu.VMEM((CHUNK, f1), jnp.float32),
            pltpu.VMEM((CHUNK, 128), jnp.float32),
            pltpu.SemaphoreType.DMA,
            pltpu.SemaphoreType.DMA,
            pltpu.SemaphoreType.DMA,
            pltpu.SemaphoreType.DMA,
        ],
    )
    def k(idx_hbm, t0, t1, t2, o0, o1, o2, idx_v, r0a, r1a, r2a, r0b, r1b,
          r2b, sga, sgb, swa, swb):
        wid = lax.axis_index("s") * NC + lax.axis_index("c")
        base0 = wid * b_per_w
        pltpu.sync_copy(idx_hbm.at[pl.ds(base0, b_per_w)], idx_v)

        def start_gather(c, r0, r1, r2, sg):
            iv = idx_v.at[pl.ds(c * CHUNK, CHUNK)]
            pltpu.async_copy(t0.at[iv], r0, sg)
            pltpu.async_copy(t1.at[iv], r1, sg)
            pltpu.async_copy(t2.at[iv], r2, sg)

        def wait_gather(r0, r1, r2, sg):
            iv = idx_v.at[pl.ds(0, CHUNK)]
            pltpu.make_async_copy(t0.at[iv], r0, sg).wait()
            pltpu.make_async_copy(t1.at[iv], r1, sg).wait()
            pltpu.make_async_copy(t2.at[iv], r2, sg).wait()

        def start_wb(c, r0, r1, r2, sw):
            base = base0 + c * CHUNK
            pltpu.async_copy(r0, o0.at[pl.ds(base, CHUNK)], sw)
            pltpu.async_copy(r1, o1.at[pl.ds(base, CHUNK)], sw)
            pltpu.async_copy(r2, o2.at[pl.ds(base, CHUNK)], sw)

        def wait_wb(r0, r1, r2, sw):
            pltpu.make_async_copy(r0, o0.at[pl.ds(0, CHUNK)], sw).wait()
            pltpu.make_async_copy(r1, o1.at[pl.ds(0, CHUNK)], sw).wait()
            pltpu.make_async_copy(r2, o2.at[pl.ds(0, CHUNK)], sw).wait()

        start_gather(0, r0a, r1a, r2a, sga)

        @pl.loop(0, n_chunks, step=2)
        def _(c):
            start_gather(c + 1, r0b, r1b, r2b, sgb)
            wait_gather(r0a, r1a, r2a, sga)
            start_wb(c, r0a, r1a, r2a, swa)
            wait_wb(r0a, r1a, r2a, swa)

            @pl.when(c + 2 < n_chunks)
            def _():
                start_gather(c + 2, r0a, r1a, r2a, sga)

            wait_gather(r0b, r1b, r2b, sgb)
            start_wb(c + 1, r0b, r1b, r2b, swb)
            wait_wb(r0b, r1b, r2b, swb)

    return k(idx, e0, e1, e2p)


N_SUB = 8


def _tc_body(*refs):
    g0, g1, g2p, w0, w1, w2, b0, b1, b2, aw, ab, pr, lng, lnb = refs[-15:-1]
    out = refs[-1]
    rows = T_BLK // N_SUB

    def chain(r0):
        sl = pl.ds(r0, rows)
        h0 = jnp.dot(g0[sl, :], w0[...],
                     preferred_element_type=jnp.float32) + b0[...]
        h1 = jnp.dot(g1[sl, :], w1[...],
                     preferred_element_type=jnp.float32) + b1[...]
        h2 = jnp.dot(g2p[sl, :64], w2[...],
                     preferred_element_type=jnp.float32) + b2[...]
        s = (
            jnp.dot(jnp.tanh(h0), aw[0:H, :],
                    preferred_element_type=jnp.float32)
            + jnp.dot(jnp.tanh(h1), aw[H:2 * H, :],
                      preferred_element_type=jnp.float32)
            + jnp.dot(jnp.tanh(h2), aw[2 * H:3 * H, :],
                      preferred_element_type=jnp.float32)
            + ab[...]
        )
        e = jnp.exp(s)
        p = e / jnp.sum(e, axis=1, keepdims=True)
        x = h0 * p[:, 0:1] + h1 * p[:, 1:2] + h2 * p[:, 2:3] \
            + pr[sl, :]
        mu = jnp.mean(x, axis=1, keepdims=True)
        xc = x - mu
        var = jnp.mean(xc * xc, axis=1, keepdims=True)
        return xc * lax.rsqrt(var + EPS) * lng[...] + lnb[...]

    ys = [chain(i * rows) for i in range(N_SUB)]
    seqs = rows // 50
    for i in range(N_SUB):
        out[i * seqs:(i + 1) * seqs] = ys[i].reshape(seqs, 50, H)


def _tc_fuse(g0, g1, g2p, W0, W1, W2, b0, b1, b2, attn_W, attn_b, posrole,
             ln_g, ln_b, out_buf=None, blk_off=0, out_rows=None):
    f0, f1, f2 = g0.shape[1], g1.shape[1], W2.shape[0]
    n_tok = g0.shape[0]
    if out_rows is None:
        out_rows = n_tok // 50
    blk = lambda i: (i, 0)
    rep = lambda i: (0, 0)
    extra_in, extra_specs, alias = (), [], {}
    if out_buf is not None:
        extra_in = (out_buf,)
        extra_specs = [pl.BlockSpec(memory_space=pl.ANY)]
        alias = {0: 0}
    return pl.pallas_call(
        _tc_body,
        grid=(n_tok // T_BLK,),
        input_output_aliases=alias,
        in_specs=extra_specs + [
            pl.BlockSpec((T_BLK, f0), blk),
            pl.BlockSpec((T_BLK, f1), blk),
            pl.BlockSpec((T_BLK, 128), blk),
            pl.BlockSpec((f0, H), rep),
            pl.BlockSpec((f1, H), rep),
            pl.BlockSpec((f2, H), rep),
            pl.BlockSpec((1, H), rep),
            pl.BlockSpec((1, H), rep),
            pl.BlockSpec((1, H), rep),
            pl.BlockSpec((3 * H, 3), rep),
            pl.BlockSpec((1, 3), rep),
            pl.BlockSpec((T_BLK, H), rep),
            pl.BlockSpec((1, H), rep),
            pl.BlockSpec((1, H), rep),
        ],
        out_specs=pl.BlockSpec((T_BLK // 50, 50, H),
                               lambda i: (i + blk_off, 0, 0)),
        out_shape=jax.ShapeDtypeStruct((out_rows, 50, H), jnp.float32),
    )(*extra_in, g0, g1, g2p, W0, W1, W2, b0, b1, b2, attn_W, attn_b,
      posrole, ln_g, ln_b)


def kernel(node_ids, emb0, emb1, emb2, W0, W1, W2, b0, b1, b2, pos_table,
           role_table, attn_W, attn_b, ln_g, ln_b):
    B, S = node_ids.shape
    idx = node_ids.reshape(-1).astype(jnp.int32)
    e2p = jnp.pad(emb2, ((0, 0), (0, 128 - emb2.shape[1])))

    role_ids = jnp.ones((S,), dtype=jnp.int32).at[0].set(0)
    posrole = pos_table[:S] + role_table[role_ids]
    posrole = jnp.tile(posrole, (T_BLK // S, 1))

    b0r, b1r, b2r = b0.reshape(1, H), b1.reshape(1, H), b2.reshape(1, H)
    abr = attn_b.reshape(1, 3)
    lngr, lnbr = ln_g.reshape(1, H), ln_b.reshape(1, H)

    out = None
    blks_per_slice = S_TOK // T_BLK
    for si in range(N_SLICES):
        isl = lax.slice(idx, (si * S_TOK,), ((si + 1) * S_TOK,))
        g0, g1, g2p = _sc_gather(isl, emb0, emb1, e2p)
        out = _tc_fuse(g0, g1, g2p, W0, W1, W2, b0r, b1r, b2r,
                       attn_W, abr, posrole, lngr, lnbr,
                       out_buf=out, blk_off=si * blks_per_slice,
                       out_rows=B)
    return out
